# baseline (speedup 1.0000x reference)
import functools

import jax
import jax.numpy as jnp
from jax import lax
from jax.experimental import pallas as pl
from jax.experimental.pallas import tpu as pltpu
from jax.experimental.pallas import tpu_sc as plsc

VOCAB = 1000000
EMBED = 64
BATCH = 16384

NC = 2
NS = 16
NW = NC * NS
B_PER_W = BATCH // NW
CHUNK = 128
NCHUNK = B_PER_W // CHUNK
PASS_ROWS = 128
NPASS = B_PER_W // PASS_ROWS
TRANS_W = 32768
HALF_W = TRANS_W // 2
NSTEPS = (VOCAB + TRANS_W - 1) // TRANS_W


def _tc_transpose(t_ref, o_ref):
    x = t_ref[...]
    xx = jnp.concatenate([x[:, :HALF_W], x[:, HALF_W:]], axis=0)
    o_ref[...] = xx.T


def _convert(table_t):
    return pl.pallas_call(
        _tc_transpose,
        grid=(NSTEPS,),
        in_specs=[pl.BlockSpec((EMBED, TRANS_W), lambda i: (0, i))],
        out_specs=pl.BlockSpec((HALF_W, 2 * EMBED), lambda i: (i, 0)),
        out_shape=jax.ShapeDtypeStruct((NSTEPS * HALF_W, 2 * EMBED),
                                       jnp.float32),
    )(table_t)


def _sc_gather_rows(target_r, in_tbl):
    @functools.partial(
        pl.kernel,
        mesh=plsc.VectorSubcoreMesh(core_axis_name="c", subcore_axis_name="s"),
        out_type=jax.ShapeDtypeStruct((BATCH, 2 * EMBED), jnp.float32),
        scratch_types=[
            pltpu.VMEM((NCHUNK, CHUNK), jnp.int32),
            pltpu.VMEM((NCHUNK, CHUNK), jnp.int32),
            pltpu.VMEM((B_PER_W, 2 * EMBED), jnp.float32),
            pltpu.SemaphoreType.DMA,
        ],
        compiler_params=pltpu.CompilerParams(needs_layout_passes=False),
    )
    def body(tgt_hbm, ine_hbm, out_hbm, idx_t, row_t, rows, sem):
        wid = lax.axis_index("s") * NC + lax.axis_index("c")
        pltpu.sync_copy(tgt_hbm.at[wid], idx_t)

        def to_row(v):
            return ((jax.lax.shift_right_logical(v, 15) * HALF_W)
                    + (v & (HALF_W - 1)))

        for j in range(NCHUNK):
            for k in range(CHUNK // 16):
                sl = pl.ds(k * 16, 16)
                row_t[j, sl] = to_row(idx_t[j, sl])

        copies = [
            pltpu.async_copy(ine_hbm.at[row_t.at[j]],
                             rows.at[pl.ds(j * CHUNK, CHUNK)], sem)
            for j in range(NCHUNK)
        ]
        for c in copies:
            c.wait()
        pltpu.sync_copy(rows, out_hbm.at[pl.ds(wid * B_PER_W, B_PER_W)])

    return body(target_r, in_tbl)


def _sc_scores(target_r, context_r, rows_t_all, out_tbl):
    @functools.partial(
        pl.kernel,
        mesh=plsc.VectorSubcoreMesh(core_axis_name="c", subcore_axis_name="s"),
        out_type=jax.ShapeDtypeStruct((BATCH,), jnp.float32),
        scratch_types=[
            pltpu.VMEM((NCHUNK, CHUNK), jnp.int32),
            pltpu.VMEM((NCHUNK, CHUNK), jnp.int32),
            pltpu.VMEM((NCHUNK, CHUNK), jnp.int32),
            pltpu.VMEM((2, PASS_ROWS, 2 * EMBED), jnp.float32),
            pltpu.VMEM((2, PASS_ROWS, 2 * EMBED), jnp.float32),
            pltpu.VMEM((B_PER_W,), jnp.float32),
            pltpu.SemaphoreType.DMA,
        ],
        compiler_params=pltpu.CompilerParams(needs_layout_passes=False),
    )
    def body(tgt_hbm, ctx_hbm, trows_hbm, oute_hbm, out_hbm,
             idx_t, idx_c, row_c, rows_t, rows_c, score, sem):
        wid = lax.axis_index("s") * NC + lax.axis_index("c")

        pltpu.sync_copy(tgt_hbm.at[wid], idx_t)
        pltpu.sync_copy(ctx_hbm.at[wid], idx_c)

        def to_row(v):
            return ((jax.lax.shift_right_logical(v, 15) * HALF_W)
                    + (v & (HALF_W - 1)))

        for j in range(NCHUNK):
            for k in range(CHUNK // 16):
                sl = pl.ds(k * 16, 16)
                row_c[j, sl] = to_row(idx_c[j, sl])

        lanes = lax.iota(jnp.int32, 16)

        base = wid * B_PER_W

        def fire(p):
            return [
                pltpu.async_copy(
                    trows_hbm.at[pl.ds(base + p * PASS_ROWS, PASS_ROWS)],
                    rows_t.at[p % 2], sem),
                pltpu.async_copy(oute_hbm.at[row_c.at[p]],
                                 rows_c.at[p % 2], sem),
            ]

        pending = fire(0)
        for p in range(NPASS):
            for c in pending:
                c.wait()
            if p + 1 < NPASS:
                pending = fire(p + 1)
            b = p % 2
            bvec = jnp.full((16,), b, dtype=jnp.int32)

            for g in range(PASS_ROWS // 16):
                o = g * 16
                tv = idx_t[p, pl.ds(o, 16)]
                cv = idx_c[p, pl.ds(o, 16)]
                off_t = (jax.lax.shift_right_logical(tv, 14) & 1) * EMBED
                off_c = (jax.lax.shift_right_logical(cv, 14) & 1) * EMBED
                rvec = g * 16 + lanes

                def inner(c, acc, off_t=off_t, off_c=off_c, rvec=rvec,
                          bvec=bvec):
                    t = plsc.load_gather(rows_t, [bvec, rvec, off_t + c])
                    u = plsc.load_gather(rows_c, [bvec, rvec, off_c + c])
                    return acc + t * u

                acc = lax.fori_loop(0, EMBED, inner,
                                    jnp.zeros((16,), jnp.float32))
                score[pl.ds(p * PASS_ROWS + g * 16, 16)] = acc

        pltpu.sync_copy(score, out_hbm.at[pl.ds(wid * B_PER_W, B_PER_W)])

    return body(target_r, context_r, rows_t_all, out_tbl)


def _tc_log_softmax(s_ref, o_ref):
    s = s_ref[...]
    m = jnp.max(s)
    lse = jnp.log(jnp.sum(jnp.exp(s - m))) + m
    o_ref[...] = s - lse


def kernel(target, context, in_embed, out_embed):
    target_r = target.astype(jnp.int32).reshape(NW, NCHUNK, CHUNK)
    context_r = context.astype(jnp.int32).reshape(NW, NCHUNK, CHUNK)
    in_tbl = _convert(in_embed.T)
    rows_t_all = _sc_gather_rows(target_r, in_tbl)
    out_tbl = _convert(out_embed.T)
    scores = _sc_scores(target_r, context_r, rows_t_all, out_tbl)
    log_probs = pl.pallas_call(
        _tc_log_softmax,
        out_shape=jax.ShapeDtypeStruct((128, 128), jnp.float32),
    )(scores.reshape(128, 128))
    return log_probs.reshape(-1)


# speedup vs baseline: 1.0121x; 1.0121x over previous
import functools

import jax
import jax.numpy as jnp
from jax import lax
from jax.experimental import pallas as pl
from jax.experimental.pallas import tpu as pltpu
from jax.experimental.pallas import tpu_sc as plsc

VOCAB = 1000000
EMBED = 64
BATCH = 16384

NC = 2
NS = 16
NW = NC * NS
B_PER_W = BATCH // NW
CHUNK = 128
NCHUNK = B_PER_W // CHUNK
PASS_ROWS = 128
NPASS = B_PER_W // PASS_ROWS
TRANS_W = 32768
HALF_W = TRANS_W // 2
NSTEPS = (VOCAB + TRANS_W - 1) // TRANS_W


def _tc_transpose(t_ref, o_ref):
    x = t_ref[...]
    xx = jnp.concatenate([x[:, :HALF_W], x[:, HALF_W:]], axis=0)
    o_ref[...] = xx.T


def _convert(table_t):
    return pl.pallas_call(
        _tc_transpose,
        grid=(NSTEPS,),
        in_specs=[pl.BlockSpec((EMBED, TRANS_W), lambda i: (0, i))],
        out_specs=pl.BlockSpec((HALF_W, 2 * EMBED), lambda i: (i, 0)),
        out_shape=jax.ShapeDtypeStruct((NSTEPS * HALF_W, 2 * EMBED),
                                       jnp.float32),
    )(table_t)


def _sc_scores(target_r, context_r, in_tbl, out_tbl):
    @functools.partial(
        pl.kernel,
        mesh=plsc.VectorSubcoreMesh(core_axis_name="c", subcore_axis_name="s"),
        out_type=jax.ShapeDtypeStruct((BATCH,), jnp.float32),
        scratch_types=[
            pltpu.VMEM((NCHUNK, CHUNK), jnp.int32),
            pltpu.VMEM((NCHUNK, CHUNK), jnp.int32),
            pltpu.VMEM((NCHUNK, CHUNK), jnp.int32),
            pltpu.VMEM((NCHUNK, CHUNK), jnp.int32),
            pltpu.VMEM((2, PASS_ROWS, 2 * EMBED), jnp.float32),
            pltpu.VMEM((2, PASS_ROWS, 2 * EMBED), jnp.float32),
            pltpu.VMEM((B_PER_W,), jnp.float32),
            pltpu.SemaphoreType.DMA,
        ],
        compiler_params=pltpu.CompilerParams(needs_layout_passes=False),
    )
    def body(tgt_hbm, ctx_hbm, ine_hbm, oute_hbm, out_hbm,
             idx_t, idx_c, row_t, row_c, rows_t, rows_c, score, sem):
        wid = lax.axis_index("s") * NC + lax.axis_index("c")

        pltpu.sync_copy(tgt_hbm.at[wid], idx_t)
        pltpu.sync_copy(ctx_hbm.at[wid], idx_c)

        def to_row(v):
            return ((jax.lax.shift_right_logical(v, 15) * HALF_W)
                    + (v & (HALF_W - 1)))

        for j in range(NCHUNK):
            for k in range(CHUNK // 16):
                sl = pl.ds(k * 16, 16)
                row_t[j, sl] = to_row(idx_t[j, sl])
                row_c[j, sl] = to_row(idx_c[j, sl])

        lanes = lax.iota(jnp.int32, 16)

        def fire(p):
            return [
                pltpu.async_copy(ine_hbm.at[row_t.at[p]],
                                 rows_t.at[p % 2], sem),
                pltpu.async_copy(oute_hbm.at[row_c.at[p]],
                                 rows_c.at[p % 2], sem),
            ]

        pending = fire(0)
        for p in range(NPASS):
            for c in pending:
                c.wait()
            if p + 1 < NPASS:
                pending = fire(p + 1)
            b = p % 2
            bvec = jnp.full((16,), b, dtype=jnp.int32)

            for g in range(PASS_ROWS // 16):
                o = g * 16
                tv = idx_t[p, pl.ds(o, 16)]
                cv = idx_c[p, pl.ds(o, 16)]
                off_t = (jax.lax.shift_right_logical(tv, 14) & 1) * EMBED
                off_c = (jax.lax.shift_right_logical(cv, 14) & 1) * EMBED
                rvec = g * 16 + lanes

                def inner(c, acc, off_t=off_t, off_c=off_c, rvec=rvec,
                          bvec=bvec):
                    t = plsc.load_gather(rows_t, [bvec, rvec, off_t + c])
                    u = plsc.load_gather(rows_c, [bvec, rvec, off_c + c])
                    return acc + t * u

                acc = lax.fori_loop(0, EMBED, inner,
                                    jnp.zeros((16,), jnp.float32))
                score[pl.ds(p * PASS_ROWS + g * 16, 16)] = acc

        pltpu.sync_copy(score, out_hbm.at[pl.ds(wid * B_PER_W, B_PER_W)])

    return body(target_r, context_r, in_tbl, out_tbl)


def _tc_log_softmax(s_ref, o_ref):
    s = s_ref[...]
    m = jnp.max(s)
    lse = jnp.log(jnp.sum(jnp.exp(s - m))) + m
    o_ref[...] = s - lse


def kernel(target, context, in_embed, out_embed):
    target_r = target.astype(jnp.int32).reshape(NW, NCHUNK, CHUNK)
    context_r = context.astype(jnp.int32).reshape(NW, NCHUNK, CHUNK)
    in_tbl = _convert(in_embed.T)
    out_tbl = _convert(out_embed.T)
    scores = _sc_scores(target_r, context_r, in_tbl, out_tbl)
    log_probs = pl.pallas_call(
        _tc_log_softmax,
        out_shape=jax.ShapeDtypeStruct((128, 128), jnp.float32),
    )(scores.reshape(128, 128))
    return log_probs.reshape(-1)
